# Initial kernel scaffold; baseline (speedup 1.0000x reference)
import jax
import jax.numpy as jnp
from jax.experimental import pallas as pl


def kernel(x, lookup_table):
    raise NotImplementedError("write your pallas kernel here")


# speedup vs baseline: 23.6308x; 23.6308x over previous
import functools
import math

import numpy as np
import jax
import jax.numpy as jnp
from jax.experimental import pallas as pl

_ALPHA, _BETA, _GAMMA = 2.0, 4.0, 8.0


def _pw_index(rp):
    rp = np.asarray(rp, dtype=np.float64)
    rp_abs = np.abs(rp)
    not_mask = rp_abs > _ALPHA
    idx = np.round(rp).astype(np.int64)
    rp_out = rp[not_mask]
    rp_abs_out = rp_abs[not_mask]
    y = (np.sign(rp_out) * np.clip(
        np.round(_ALPHA + np.log(rp_abs_out / _ALPHA)
                 / math.log(_GAMMA / _ALPHA) * (_BETA - _ALPHA)),
        None, _BETA)).astype(np.int64)
    idx[not_mask] = y
    return idx


def _quant(ids):
    uq, inv = np.unique(ids, return_inverse=True)
    return inv.reshape(ids.shape), uq.size


@functools.lru_cache(maxsize=None)
def _bucket_map(L):
    E = int(math.isqrt(L))
    assert E * E == L
    rg = np.arange(E)
    rows = np.repeat(rg[:, None], E, axis=1)
    cols = rows.T
    pos = np.stack([rows, cols], 2).reshape(E * E, 2)
    diff = pos[:, None, :] - pos[None, :, :]
    r, r_num = _quant(_pw_index(diff[:, :, 0]))
    c, c_num = _quant(_pw_index(diff[:, :, 1]))
    pid = r * c_num + c
    return pid.astype(np.int32)


def _body(tab_ref, b_ref, o_ref, *, heads, block_rows, L):
    tab = tab_ref[...]
    for s in range(block_rows):
        row = b_ref[s, :]
        iota = jax.lax.broadcasted_iota(jnp.int32, (128, L), 0)
        oh = (iota == row[None, :]).astype(jnp.float32)
        res = jnp.dot(tab, oh, preferred_element_type=jnp.float32)
        o_ref[:, s, :] = res[:heads, :]


def kernel(x, lookup_table):
    L = x.shape[2]
    H, B = lookup_table.shape
    bucket = jnp.asarray(_bucket_map(L))
    table_pad = jnp.zeros((16, 128), jnp.float32).at[:H, :B].set(lookup_table)

    block_rows = 8
    nblk = L // block_rows
    out = pl.pallas_call(
        functools.partial(_body, heads=H, block_rows=block_rows, L=L),
        grid=(nblk,),
        in_specs=[
            pl.BlockSpec((16, 128), lambda i: (0, 0)),
            pl.BlockSpec((block_rows, L), lambda i: (i, 0)),
        ],
        out_specs=pl.BlockSpec((H, block_rows, L), lambda i: (0, i, 0)),
        out_shape=jax.ShapeDtypeStruct((H, L, L), jnp.float32),
    )(table_pad, bucket)
    return out.reshape(1, H, L, L)
